# Initial kernel scaffold; baseline (speedup 1.0000x reference)
import jax
import jax.numpy as jnp
from jax import lax
from jax.experimental import pallas as pl
from jax.experimental.pallas import tpu as pltpu
from jax.experimental.pallas import tpu_sc as plsc

N = 10000
E = 320000
D = 128

NC = 2
NS = 16
NW = NC * NS
L = 16

CHUNK = 128
CPW = -(-E // (NW * CHUNK))
E_PAD = NW * CPW * CHUNK
ROWS_PER_TILE = 640
N_PAD = NS * ROWS_PER_TILE
VECS = D // L


def _sc_body(x_hbm, col_hbm, dst_hbm, w_hbm, out_hbm,
             acc, colv, dstv, wv, rows, gsem):
    cid = lax.axis_index("c")
    sid = lax.axis_index("s")
    wid = sid * NC + cid

    zero16 = jnp.zeros((L,), jnp.float32)

    @pl.loop(0, CHUNK)
    def _zero_rows(i):
        for k in range(VECS):
            rows[i, pl.ds(k * L, L)] = zero16

    for c in range(ROWS_PER_TILE // CHUNK):
        pltpu.sync_copy(rows, acc.at[pl.ds(sid * ROWS_PER_TILE + c * CHUNK, CHUNK)])
    plsc.subcore_barrier()

    @pl.loop(0, CPW)
    def _chunk(g):
        base = wid * (CPW * CHUNK) + g * CHUNK
        pltpu.sync_copy(col_hbm.at[pl.ds(base, CHUNK)], colv)
        pltpu.sync_copy(dst_hbm.at[pl.ds(base, CHUNK)], dstv)
        pltpu.sync_copy(w_hbm.at[pl.ds(base, CHUNK)], wv)
        pltpu.async_copy(x_hbm.at[colv], rows, gsem).wait()

        @pl.loop(0, CHUNK)
        def _scale(j):
            wj = wv[j]
            for k in range(VECS):
                sl = pl.ds(k * L, L)
                rows[j, sl] = rows[j, sl] * wj

        pltpu.sync_copy(rows, acc.at[dstv], add=True)

    plsc.subcore_barrier()
    out_base = cid * N_PAD + sid * ROWS_PER_TILE
    for c in range(ROWS_PER_TILE // CHUNK):
        pltpu.sync_copy(acc.at[pl.ds(sid * ROWS_PER_TILE + c * CHUNK, CHUNK)],
                        out_hbm.at[pl.ds(out_base + c * CHUNK, CHUNK)])


_sc_scatter = pl.kernel(
    _sc_body,
    out_type=jax.ShapeDtypeStruct((NC * N_PAD, D), jnp.float32),
    mesh=plsc.VectorSubcoreMesh(core_axis_name="c", subcore_axis_name="s",
                                num_cores=NC, num_subcores=NS),
    scratch_types=[
        pltpu.VMEM_SHARED((N_PAD, D), jnp.float32),
        pltpu.VMEM((CHUNK,), jnp.int32),
        pltpu.VMEM((CHUNK,), jnp.int32),
        pltpu.VMEM((CHUNK,), jnp.float32),
        pltpu.VMEM((CHUNK, D), jnp.float32),
        pltpu.SemaphoreType.DMA,
    ],
)


def _combine_body(p_ref, o_ref):
    s = p_ref[0] + p_ref[1]
    o_ref[...] = jnp.clip(s, -1000000.0, 1000000.0)


_BLK = 1024
_combine = pl.pallas_call(
    _combine_body,
    grid=(N_PAD // _BLK,),
    in_specs=[pl.BlockSpec((2, _BLK, D), lambda i: (0, i, 0))],
    out_specs=pl.BlockSpec((_BLK, D), lambda i: (i, 0)),
    out_shape=jax.ShapeDtypeStruct((N_PAD, D), jnp.float32),
)


def kernel(t, x, edge_index, edge_weight):
    del t
    pad = E_PAD - E
    col = jnp.concatenate([edge_index[1], jnp.zeros((pad,), jnp.int32)])
    dst = jnp.concatenate([edge_index[0], jnp.zeros((pad,), jnp.int32)])
    w = jnp.concatenate([edge_weight, jnp.zeros((pad,), jnp.float32)])
    partials = _sc_scatter(x, col, dst, w)
    out = _combine(partials.reshape(NC, N_PAD, D))
    return out[:N]


# speedup vs baseline: 3.3799x; 3.3799x over previous
import jax
import jax.numpy as jnp
from jax import lax
from jax.experimental import pallas as pl
from jax.experimental.pallas import tpu as pltpu
from jax.experimental.pallas import tpu_sc as plsc

N = 10000
E = 320000
D = 128

NC = 2
NS = 16
NW = NC * NS
L = 16

CHUNK = 128
CPW = -(-E // (NW * CHUNK))
E_PAD = NW * CPW * CHUNK
ROWS_PER_TILE = 640
N_PAD = NS * ROWS_PER_TILE
VECS = D // L


def _sc_body(x_hbm, col_hbm, dst_hbm, w_hbm, out_hbm,
             acc, colv, dstv, wv, rows, gsem):
    cid = lax.axis_index("c")
    sid = lax.axis_index("s")
    wid = sid * NC + cid

    zero16 = jnp.zeros((L,), jnp.float32)

    @pl.loop(0, CHUNK)
    def _zero_rows(i):
        for k in range(VECS):
            rows[i, pl.ds(k * L, L)] = zero16

    for c in range(ROWS_PER_TILE // CHUNK):
        pltpu.sync_copy(rows, acc.at[pl.ds(sid * ROWS_PER_TILE + c * CHUNK, CHUNK)])
    plsc.subcore_barrier()

    @pl.loop(0, CPW)
    def _chunk(g):
        base = wid * (CPW * CHUNK) + g * CHUNK
        pltpu.sync_copy(col_hbm.at[pl.ds(base, CHUNK)], colv)
        pltpu.sync_copy(dst_hbm.at[pl.ds(base, CHUNK)], dstv)
        pltpu.sync_copy(w_hbm.at[pl.ds(base, CHUNK)], wv)
        pltpu.async_copy(x_hbm.at[colv], rows, gsem).wait()

        @pl.loop(0, CHUNK // L)
        def _scale(jg):
            wvec = wv[pl.ds(jg * L, L)]
            for jj in range(L):
                j = jg * L + jj
                wj = wvec[jj]
                for k in range(VECS):
                    sl = pl.ds(k * L, L)
                    rows[j, sl] = rows[j, sl] * wj

        @pl.when(sid % 2 == 0)
        def _even_add():
            pltpu.sync_copy(rows, acc.at[dstv], add=True)

        plsc.subcore_barrier()

        @pl.when(sid % 2 == 1)
        def _odd_add():
            pltpu.sync_copy(rows, acc.at[dstv], add=True)

        plsc.subcore_barrier()

    plsc.subcore_barrier()
    out_base = cid * N_PAD + sid * ROWS_PER_TILE
    for c in range(ROWS_PER_TILE // CHUNK):
        pltpu.sync_copy(acc.at[pl.ds(sid * ROWS_PER_TILE + c * CHUNK, CHUNK)],
                        out_hbm.at[pl.ds(out_base + c * CHUNK, CHUNK)])


_sc_scatter = pl.kernel(
    _sc_body,
    out_type=jax.ShapeDtypeStruct((NC * N_PAD, D), jnp.float32),
    mesh=plsc.VectorSubcoreMesh(core_axis_name="c", subcore_axis_name="s",
                                num_cores=NC, num_subcores=NS),
    scratch_types=[
        pltpu.VMEM_SHARED((N_PAD, D), jnp.float32),
        pltpu.VMEM((CHUNK,), jnp.int32),
        pltpu.VMEM((CHUNK,), jnp.int32),
        pltpu.VMEM((CHUNK,), jnp.float32),
        pltpu.VMEM((CHUNK, D), jnp.float32),
        pltpu.SemaphoreType.DMA,
    ],
)


def _combine_body(p_ref, o_ref):
    s = p_ref[0] + p_ref[1]
    o_ref[...] = jnp.clip(s, -1000000.0, 1000000.0)


_BLK = 1024
_combine = pl.pallas_call(
    _combine_body,
    grid=(N_PAD // _BLK,),
    in_specs=[pl.BlockSpec((2, _BLK, D), lambda i: (0, i, 0))],
    out_specs=pl.BlockSpec((_BLK, D), lambda i: (i, 0)),
    out_shape=jax.ShapeDtypeStruct((N_PAD, D), jnp.float32),
)


def kernel(t, x, edge_index, edge_weight):
    del t
    pad = E_PAD - E
    col = jnp.concatenate([edge_index[1], jnp.zeros((pad,), jnp.int32)])
    dst = jnp.concatenate([edge_index[0], jnp.zeros((pad,), jnp.int32)])
    w = jnp.concatenate([edge_weight, jnp.zeros((pad,), jnp.float32)])
    partials = _sc_scatter(x, col, dst, w)
    out = _combine(partials.reshape(NC, N_PAD, D))
    return out[:N]


# speedup vs baseline: 3.4923x; 1.0333x over previous
import jax
import jax.numpy as jnp
from jax import lax
from jax.experimental import pallas as pl
from jax.experimental.pallas import tpu as pltpu
from jax.experimental.pallas import tpu_sc as plsc

N = 10000
E = 320000
D = 128

NC = 2
NS = 16
NW = NC * NS
L = 16

CHUNK = 128
CPW = 80
E_PAD = NW * CPW * CHUNK
G_TOTAL = NW * CPW
ROWS_PER_TILE = 640
N_PAD = NS * ROWS_PER_TILE
VECS = D // L


def _scale_rows(rows, wbuf):
    @pl.loop(0, CHUNK // L)
    def _scale(jg):
        wvec = wbuf[pl.ds(jg * L, L)]
        for jj in range(L):
            j = jg * L + jj
            wj = wvec[jj]
            for k in range(VECS):
                sl = pl.ds(k * L, L)
                rows[j, sl] = rows[j, sl] * wj


def _phased_add(sid, rows, acc, ibuf):
    @pl.when(sid % 2 == 0)
    def _even_add():
        pltpu.sync_copy(rows, acc.at[ibuf.at[1]], add=True)

    plsc.subcore_barrier()

    @pl.when(sid % 2 == 1)
    def _odd_add():
        pltpu.sync_copy(rows, acc.at[ibuf.at[1]], add=True)

    plsc.subcore_barrier()


def _sc_body(x_hbm, idx_hbm, w_hbm, out_hbm,
             acc, ibuf0, ibuf1, wbuf0, wbuf1, rows0, rows1, gsem, isem, wsem):
    cid = lax.axis_index("c")
    sid = lax.axis_index("s")
    wid = sid * NC + cid
    wbase = wid * CPW

    ibufs = (ibuf0, ibuf1)
    wbufs = (wbuf0, wbuf1)
    rowss = (rows0, rows1)

    zero16 = jnp.zeros((L,), jnp.float32)

    @pl.loop(0, CHUNK)
    def _zero_rows(i):
        for k in range(VECS):
            rows0[i, pl.ds(k * L, L)] = zero16

    for c in range(ROWS_PER_TILE // CHUNK):
        pltpu.sync_copy(rows0, acc.at[pl.ds(sid * ROWS_PER_TILE + c * CHUNK, CHUNK)])
    plsc.subcore_barrier()

    pltpu.sync_copy(idx_hbm.at[wbase], ibuf0)
    pltpu.sync_copy(w_hbm.at[wbase], wbuf0)
    pltpu.async_copy(x_hbm.at[ibuf0.at[0]], rows0, gsem.at[0])
    pltpu.async_copy(idx_hbm.at[wbase + 1], ibuf1, isem.at[1])
    pltpu.async_copy(w_hbm.at[wbase + 1], wbuf1, wsem.at[1])

    @pl.loop(0, CPW, step=2)
    def _chunk(g0):
        for par in range(2):
            g = g0 + par
            p, q = par, 1 - par
            pltpu.make_async_copy(x_hbm.at[ibufs[p].at[0]], rowss[p],
                                  gsem.at[p]).wait()

            @pl.when(g + 1 < CPW)
            def _next_gather():
                pltpu.make_async_copy(idx_hbm.at[wbase + g + 1], ibufs[q],
                                      isem.at[q]).wait()
                pltpu.async_copy(x_hbm.at[ibufs[q].at[0]], rowss[q], gsem.at[q])

            @pl.when(g > 0)
            def _wait_w():
                pltpu.make_async_copy(w_hbm.at[wbase + g], wbufs[p],
                                      wsem.at[p]).wait()

            _scale_rows(rowss[p], wbufs[p])
            _phased_add(sid, rowss[p], acc, ibufs[p])

            @pl.when(g + 2 < CPW)
            def _next_idx():
                pltpu.async_copy(idx_hbm.at[wbase + g + 2], ibufs[p], isem.at[p])
                pltpu.async_copy(w_hbm.at[wbase + g + 2], wbufs[p], wsem.at[p])

    plsc.subcore_barrier()
    out_base = cid * N_PAD + sid * ROWS_PER_TILE
    for c in range(ROWS_PER_TILE // CHUNK):
        pltpu.sync_copy(acc.at[pl.ds(sid * ROWS_PER_TILE + c * CHUNK, CHUNK)],
                        out_hbm.at[pl.ds(out_base + c * CHUNK, CHUNK)])


_sc_scatter = pl.kernel(
    _sc_body,
    out_type=jax.ShapeDtypeStruct((NC * N_PAD, D), jnp.float32),
    mesh=plsc.VectorSubcoreMesh(core_axis_name="c", subcore_axis_name="s",
                                num_cores=NC, num_subcores=NS),
    scratch_types=[
        pltpu.VMEM_SHARED((N_PAD, D), jnp.float32),
        pltpu.VMEM((2, CHUNK), jnp.int32),
        pltpu.VMEM((2, CHUNK), jnp.int32),
        pltpu.VMEM((CHUNK,), jnp.float32),
        pltpu.VMEM((CHUNK,), jnp.float32),
        pltpu.VMEM((CHUNK, D), jnp.float32),
        pltpu.VMEM((CHUNK, D), jnp.float32),
        pltpu.SemaphoreType.DMA((2,)),
        pltpu.SemaphoreType.DMA((2,)),
        pltpu.SemaphoreType.DMA((2,)),
    ],
)


def _combine_body(p_ref, o_ref):
    s = p_ref[0] + p_ref[1]
    o_ref[...] = jnp.clip(s, -1000000.0, 1000000.0)


_BLK = 1024
_combine = pl.pallas_call(
    _combine_body,
    grid=(N_PAD // _BLK,),
    in_specs=[pl.BlockSpec((2, _BLK, D), lambda i: (0, i, 0))],
    out_specs=pl.BlockSpec((_BLK, D), lambda i: (i, 0)),
    out_shape=jax.ShapeDtypeStruct((N_PAD, D), jnp.float32),
)


def kernel(t, x, edge_index, edge_weight):
    del t
    pad = E_PAD - E
    col = jnp.concatenate([edge_index[1], jnp.zeros((pad,), jnp.int32)])
    dst = jnp.concatenate([edge_index[0], jnp.zeros((pad,), jnp.int32)])
    w = jnp.concatenate([edge_weight, jnp.zeros((pad,), jnp.float32)])
    packed = jnp.stack([col, dst]).reshape(2, G_TOTAL, CHUNK)
    packed = packed.transpose(1, 0, 2)
    partials = _sc_scatter(x, packed, w.reshape(G_TOTAL, CHUNK))
    out = _combine(partials.reshape(NC, N_PAD, D))
    return out[:N]


# speedup vs baseline: 3.5451x; 1.0151x over previous
import jax
import jax.numpy as jnp
from jax import lax
from jax.experimental import pallas as pl
from jax.experimental.pallas import tpu as pltpu
from jax.experimental.pallas import tpu_sc as plsc

N = 10000
E = 320000
D = 128

NC = 2
NS = 16
NW = NC * NS
L = 16

CHUNK = 128
CPW = 80
E_PAD = NW * CPW * CHUNK
G_TOTAL = NW * CPW
ROWS_PER_TILE = 640
N_PAD = NS * ROWS_PER_TILE
VECS = D // L


def _scale_rows(rows, wbuf):
    @pl.loop(0, CHUNK // L)
    def _scale(jg):
        wvec = wbuf[pl.ds(jg * L, L)]
        for jj in range(L):
            j = jg * L + jj
            wj = wvec[jj]
            for k in range(VECS):
                sl = pl.ds(k * L, L)
                rows[j, sl] = rows[j, sl] * wj


def _sc_body(x_hbm, idx_hbm, w_hbm, out_hbm,
             acc, ibuf0, ibuf1, wbuf0, wbuf1, rows0, rows1, gsem, isem, wsem):
    cid = lax.axis_index("c")
    sid = lax.axis_index("s")
    wid = sid * NC + cid
    wbase = wid * CPW

    ibufs = (ibuf0, ibuf1)
    wbufs = (wbuf0, wbuf1)
    rowss = (rows0, rows1)

    zero16 = jnp.zeros((L,), jnp.float32)

    @pl.loop(0, CHUNK)
    def _zero_rows(i):
        for k in range(VECS):
            rows0[i, pl.ds(k * L, L)] = zero16

    for c in range(ROWS_PER_TILE // CHUNK):
        pltpu.sync_copy(rows0, acc.at[pl.ds(sid * ROWS_PER_TILE + c * CHUNK, CHUNK)])
    plsc.subcore_barrier()

    pltpu.sync_copy(idx_hbm.at[wbase], ibuf0)
    pltpu.sync_copy(w_hbm.at[wbase], wbuf0)
    pltpu.async_copy(x_hbm.at[ibuf0.at[0]], rows0, gsem.at[0])
    pltpu.async_copy(idx_hbm.at[wbase + 1], ibuf1, isem.at[1])
    pltpu.async_copy(w_hbm.at[wbase + 1], wbuf1, wsem.at[1])

    @pl.loop(0, CPW, step=2)
    def _chunk(g0):
        for par in range(2):
            g = g0 + par
            p, q = par, 1 - par
            pltpu.make_async_copy(x_hbm.at[ibufs[p].at[0]], rowss[p],
                                  gsem.at[p]).wait()

            @pl.when(g + 1 < CPW)
            def _next_gather():
                pltpu.make_async_copy(idx_hbm.at[wbase + g + 1], ibufs[q],
                                      isem.at[q]).wait()
                pltpu.async_copy(x_hbm.at[ibufs[q].at[0]], rowss[q], gsem.at[q])

            @pl.when(g > 0)
            def _wait_w():
                pltpu.make_async_copy(w_hbm.at[wbase + g], wbufs[p],
                                      wsem.at[p]).wait()

            even = sid % 2 == 0

            @pl.when(even)
            def _even_scale():
                _scale_rows(rowss[p], wbufs[p])

            plsc.subcore_barrier()

            @pl.when(even)
            def _even_add():
                pltpu.sync_copy(rowss[p], acc.at[ibufs[p].at[1]], add=True)

            @pl.when(jnp.logical_not(even))
            def _odd_scale():
                _scale_rows(rowss[p], wbufs[p])

            plsc.subcore_barrier()

            @pl.when(jnp.logical_not(even))
            def _odd_add():
                pltpu.sync_copy(rowss[p], acc.at[ibufs[p].at[1]], add=True)

            @pl.when(g + 2 < CPW)
            def _next_idx():
                pltpu.async_copy(idx_hbm.at[wbase + g + 2], ibufs[p], isem.at[p])
                pltpu.async_copy(w_hbm.at[wbase + g + 2], wbufs[p], wsem.at[p])

    plsc.subcore_barrier()
    out_base = cid * N_PAD + sid * ROWS_PER_TILE
    for c in range(ROWS_PER_TILE // CHUNK):
        pltpu.sync_copy(acc.at[pl.ds(sid * ROWS_PER_TILE + c * CHUNK, CHUNK)],
                        out_hbm.at[pl.ds(out_base + c * CHUNK, CHUNK)])


_sc_scatter = pl.kernel(
    _sc_body,
    out_type=jax.ShapeDtypeStruct((NC * N_PAD, D), jnp.float32),
    mesh=plsc.VectorSubcoreMesh(core_axis_name="c", subcore_axis_name="s",
                                num_cores=NC, num_subcores=NS),
    scratch_types=[
        pltpu.VMEM_SHARED((N_PAD, D), jnp.float32),
        pltpu.VMEM((2, CHUNK), jnp.int32),
        pltpu.VMEM((2, CHUNK), jnp.int32),
        pltpu.VMEM((CHUNK,), jnp.float32),
        pltpu.VMEM((CHUNK,), jnp.float32),
        pltpu.VMEM((CHUNK, D), jnp.float32),
        pltpu.VMEM((CHUNK, D), jnp.float32),
        pltpu.SemaphoreType.DMA((2,)),
        pltpu.SemaphoreType.DMA((2,)),
        pltpu.SemaphoreType.DMA((2,)),
    ],
)


def _combine_body(p_ref, o_ref):
    s = p_ref[0] + p_ref[1]
    o_ref[...] = jnp.clip(s, -1000000.0, 1000000.0)


_BLK = 1024
_combine = pl.pallas_call(
    _combine_body,
    grid=(N_PAD // _BLK,),
    in_specs=[pl.BlockSpec((2, _BLK, D), lambda i: (0, i, 0))],
    out_specs=pl.BlockSpec((_BLK, D), lambda i: (i, 0)),
    out_shape=jax.ShapeDtypeStruct((N_PAD, D), jnp.float32),
)


def kernel(t, x, edge_index, edge_weight):
    del t
    pad = E_PAD - E
    col = jnp.concatenate([edge_index[1], jnp.zeros((pad,), jnp.int32)])
    dst = jnp.concatenate([edge_index[0], jnp.zeros((pad,), jnp.int32)])
    w = jnp.concatenate([edge_weight, jnp.zeros((pad,), jnp.float32)])
    packed = jnp.stack([col, dst]).reshape(2, G_TOTAL, CHUNK)
    packed = packed.transpose(1, 0, 2)
    partials = _sc_scatter(x, packed, w.reshape(G_TOTAL, CHUNK))
    out = _combine(partials.reshape(NC, N_PAD, D))
    return out[:N]
